# baseline (speedup 1.0000x reference)
import jax
import jax.numpy as jnp
from jax.experimental import pallas as pl
from jax.experimental.pallas import tpu as pltpu

_NUM_HEADS = 16
_THETA = 10000.0
_EPS = 1e-5
_NEG = float(jnp.finfo(jnp.float32).min)
_LOG2E = 1.4426950408889634
_DN_TB = (((1,), (1,)), ((), ()))

_BM_QKV = 512
_BQ = 256
_BM_FFN = 128
_BF = 4096
_NSPLIT = 8


def _qkv_body(x_ref, wq_ref, wk_ref, wv_ref, g1_ref, cos_ref, sin_ref,
              q_ref, k_ref, v_ref):
    x = x_ref[...]
    ms = jnp.mean(x * x, axis=1, keepdims=True)
    h = (x * jax.lax.rsqrt(ms + _EPS) * g1_ref[...]).astype(jnp.bfloat16)
    q = jax.lax.dot_general(h, wq_ref[...], _DN_TB,
                            preferred_element_type=jnp.float32)
    k = jax.lax.dot_general(h, wk_ref[...], _DN_TB,
                            preferred_element_type=jnp.float32)
    v = jax.lax.dot_general(h, wv_ref[...], _DN_TB,
                            preferred_element_type=jnp.float32)
    reps = q.shape[1] // cos_ref.shape[1]
    c = cos_ref[...]
    s = sin_ref[...]
    cf = jnp.concatenate([c] * reps, axis=1)
    sf = jnp.concatenate([s] * reps, axis=1)
    lane = jax.lax.broadcasted_iota(jnp.int32, q.shape, 1)
    even = (lane % 2) == 0

    def rope(t):
        fwd = pltpu.roll(t, 1, axis=1)
        bwd = pltpu.roll(t, t.shape[1] - 1, axis=1)
        return t * cf + jnp.where(even, -bwd, fwd) * sf

    q_ref[...] = (rope(q) * (0.125 * _LOG2E)).astype(jnp.bfloat16)
    k_ref[...] = rope(k).astype(jnp.bfloat16)
    v_ref[...] = v.astype(jnp.bfloat16)


def _qkv(x2d, wq, wk, wv, g1, cos64, sin64, *, interpret=False):
    R, D = x2d.shape
    S = cos64.shape[0]
    bm = _BM_QKV
    nr = R // bm
    npos = S // bm
    w_spec = pl.BlockSpec((D, D), lambda i: (0, 0))
    row_spec = pl.BlockSpec((bm, D), lambda i: (i, 0))
    tab_spec = pl.BlockSpec((bm, cos64.shape[1]), lambda i: (i % npos, 0))
    return pl.pallas_call(
        _qkv_body,
        grid=(nr,),
        in_specs=[row_spec, w_spec, w_spec, w_spec,
                  pl.BlockSpec((1, D), lambda i: (0, 0)),
                  tab_spec, tab_spec],
        out_specs=[row_spec, row_spec, row_spec],
        out_shape=[jax.ShapeDtypeStruct((R, D), jnp.bfloat16)] * 3,
        compiler_params=pltpu.CompilerParams(
            dimension_semantics=("parallel",),
            vmem_limit_bytes=50 * 1024 * 1024,
        ),
        name="qkv_rmsnorm_rope",
        interpret=interpret,
    )(x2d, wq, wk, wv, g1, cos64, sin64)


def _attn_piece_body(q_ref, k_ref, v_ref, o_ref, *, row0):
    i = pl.program_id(2)
    bq = q_ref.shape[1]
    Sk = k_ref.shape[1]
    q = q_ref[0]
    k = k_ref[0]
    v = v_ref[0]
    rows = row0 + i * bq + jax.lax.broadcasted_iota(jnp.int32, (bq, Sk), 0)
    cols = jax.lax.broadcasted_iota(jnp.int32, (bq, Sk), 1)
    mask = cols <= rows
    halves = []
    for hh in range(2):
        sl = slice(hh * 64, hh * 64 + 64)
        s = jax.lax.dot_general(q[:, sl], k[:, sl], _DN_TB,
                                preferred_element_type=jnp.float32)
        p = jnp.exp2(jnp.where(mask, s, _NEG))
        l = jnp.sum(p, axis=1, keepdims=True)
        oh = jnp.dot(p.astype(jnp.bfloat16), v[:, sl],
                     preferred_element_type=jnp.float32)
        halves.append((oh / l).astype(jnp.bfloat16))
    o_ref[0] = jnp.concatenate(halves, axis=1)


def _attn(q, k, v, *, interpret=False):
    B, S, D = q.shape
    bq = _BQ
    hp = D // 128
    import functools
    pieces = []
    rows_per = S // _NSPLIT
    nq = rows_per // bq
    for p in range(_NSPLIT):
        Sk = (p + 1) * rows_per
        row0 = p * rows_per
        q_spec = pl.BlockSpec((1, bq, 128),
                              lambda b, h, i, _p=p, _nq=nq: (b, _p * _nq + i, h))
        kv_spec = pl.BlockSpec((1, Sk, 128), lambda b, h, i: (b, 0, h))
        o_spec = pl.BlockSpec((1, bq, 128), lambda b, h, i: (b, i, h))
        piece = pl.pallas_call(
            functools.partial(_attn_piece_body, row0=row0),
            grid=(B, hp, nq),
            in_specs=[q_spec, kv_spec, kv_spec],
            out_specs=o_spec,
            out_shape=jax.ShapeDtypeStruct((B, rows_per, D), jnp.bfloat16),
            compiler_params=pltpu.CompilerParams(
                dimension_semantics=("parallel", "arbitrary", "arbitrary"),
                vmem_limit_bytes=40 * 1024 * 1024,
            ),
            name=f"causal_attention_p{p}",
            interpret=interpret,
        )(q, k, v)
        pieces.append(piece)
    return jnp.concatenate(pieces, axis=1)


def _ffn_body(o_ref, x_ref, wo_ref, g2_ref, w1_ref, w3_ref, w2_ref,
              out_ref, h2_ref):
    j = pl.program_id(1)

    @pl.when(j == 0)
    def _():
        x1 = x_ref[...] + jax.lax.dot_general(
            o_ref[...], wo_ref[...], _DN_TB,
            preferred_element_type=jnp.float32)
        out_ref[...] = x1
        ms = jnp.mean(x1 * x1, axis=1, keepdims=True)
        h2_ref[...] = (x1 * jax.lax.rsqrt(ms + _EPS)
                       * g2_ref[...]).astype(jnp.bfloat16)

    h2 = h2_ref[...]
    a = jax.lax.dot_general(h2, w1_ref[...], _DN_TB,
                            preferred_element_type=jnp.float32)
    b = jax.lax.dot_general(h2, w3_ref[...], _DN_TB,
                            preferred_element_type=jnp.float32)
    ff = (a * jax.nn.sigmoid(a) * b).astype(jnp.bfloat16)
    out_ref[...] += jax.lax.dot_general(ff, w2_ref[...], _DN_TB,
                                        preferred_element_type=jnp.float32)


def _ffn(o2d, x2d, wo, g2, w1, w3, w2, *, interpret=False):
    R, D = x2d.shape
    F = w1.shape[0]
    bm = _BM_FFN
    bf = _BF
    nr = R // bm
    nf = F // bf
    row_spec = pl.BlockSpec((bm, D), lambda i, j: (i, 0))
    return pl.pallas_call(
        _ffn_body,
        grid=(nr, nf),
        in_specs=[row_spec, row_spec,
                  pl.BlockSpec((D, D), lambda i, j: (0, 0)),
                  pl.BlockSpec((1, D), lambda i, j: (0, 0)),
                  pl.BlockSpec((bf, D), lambda i, j: (j, 0)),
                  pl.BlockSpec((bf, D), lambda i, j: (j, 0)),
                  pl.BlockSpec((D, bf), lambda i, j: (0, j))],
        out_specs=row_spec,
        out_shape=jax.ShapeDtypeStruct((R, D), jnp.float32),
        scratch_shapes=[pltpu.VMEM((bm, D), jnp.bfloat16)],
        compiler_params=pltpu.CompilerParams(
            dimension_semantics=("parallel", "arbitrary"),
            vmem_limit_bytes=56 * 1024 * 1024,
        ),
        name="outproj_rmsnorm_swiglu",
        interpret=interpret,
    )(o2d, x2d, wo, g2, w1, w3, w2)


def kernel(x, wq, wk, wv, wo, w1, w2, w3, g1, g2, *, interpret=False):
    B, S, D = x.shape
    dh = D // _NUM_HEADS
    x2d = x.reshape(B * S, D)
    wqb = wq.astype(jnp.bfloat16)
    wkb = wk.astype(jnp.bfloat16)
    wvb = wv.astype(jnp.bfloat16)
    wob = wo.astype(jnp.bfloat16)
    w1b = w1.astype(jnp.bfloat16)
    w2b = w2.astype(jnp.bfloat16)
    w3b = w3.astype(jnp.bfloat16)
    inv = _THETA ** (-jnp.arange(0, dh, 2, dtype=jnp.float32) / dh)
    ang = jnp.arange(S, dtype=jnp.float32)[:, None] * inv[None, :]
    cos64 = jnp.repeat(jnp.cos(ang), 2, axis=1)
    sin64 = jnp.repeat(jnp.sin(ang), 2, axis=1)
    q2d, k2d, v2d = _qkv(x2d, wqb, wkb, wvb, g1.reshape(1, D), cos64, sin64,
                         interpret=interpret)
    o = _attn(q2d.reshape(B, S, D), k2d.reshape(B, S, D),
              v2d.reshape(B, S, D), interpret=interpret)
    out2d = _ffn(o.reshape(B * S, D), x2d, wob, g2.reshape(1, D),
                 w1b, w3b, w2b, interpret=interpret)
    return out2d.reshape(B, S, D)


# speedup vs baseline: 1.5219x; 1.5219x over previous
import jax
import jax.numpy as jnp
from jax.experimental import pallas as pl
from jax.experimental.pallas import tpu as pltpu

_NUM_HEADS = 16
_THETA = 10000.0
_EPS = 1e-5
_NEG = float(jnp.finfo(jnp.float32).min)
_LOG2E = 1.4426950408889634
_DN_TB = (((1,), (1,)), ((), ()))

_BM_QKV = 512
_BQ = 256
_BM_FFN = 512
_BF = 2048
_NSPLIT = 8


def _qkv_body(x_ref, wq_ref, wk_ref, wv_ref, g1_ref, cos_ref, sin_ref,
              q_ref, k_ref, v_ref):
    x = x_ref[...]
    ms = jnp.mean(x * x, axis=1, keepdims=True)
    h = (x * jax.lax.rsqrt(ms + _EPS) * g1_ref[...]).astype(jnp.bfloat16)
    q = jax.lax.dot_general(h, wq_ref[...], _DN_TB,
                            preferred_element_type=jnp.float32)
    k = jax.lax.dot_general(h, wk_ref[...], _DN_TB,
                            preferred_element_type=jnp.float32)
    v = jax.lax.dot_general(h, wv_ref[...], _DN_TB,
                            preferred_element_type=jnp.float32)
    reps = q.shape[1] // cos_ref.shape[1]
    c = cos_ref[...]
    s = sin_ref[...]
    cf = jnp.concatenate([c] * reps, axis=1)
    sf = jnp.concatenate([s] * reps, axis=1)
    lane = jax.lax.broadcasted_iota(jnp.int32, q.shape, 1)
    even = (lane % 2) == 0

    def rope(t):
        fwd = pltpu.roll(t, 1, axis=1)
        bwd = pltpu.roll(t, t.shape[1] - 1, axis=1)
        return t * cf + jnp.where(even, -bwd, fwd) * sf

    q_ref[...] = (rope(q) * (0.125 * _LOG2E)).astype(jnp.bfloat16)
    k_ref[...] = rope(k).astype(jnp.bfloat16)
    v_ref[...] = v.astype(jnp.bfloat16)


def _qkv(x2d, wq, wk, wv, g1, cos64, sin64, *, interpret=False):
    R, D = x2d.shape
    S = cos64.shape[0]
    bm = _BM_QKV
    nr = R // bm
    npos = S // bm
    w_spec = pl.BlockSpec((D, D), lambda i: (0, 0))
    row_spec = pl.BlockSpec((bm, D), lambda i: (i, 0))
    tab_spec = pl.BlockSpec((bm, cos64.shape[1]), lambda i: (i % npos, 0))
    return pl.pallas_call(
        _qkv_body,
        grid=(nr,),
        in_specs=[row_spec, w_spec, w_spec, w_spec,
                  pl.BlockSpec((1, D), lambda i: (0, 0)),
                  tab_spec, tab_spec],
        out_specs=[row_spec, row_spec, row_spec],
        out_shape=[jax.ShapeDtypeStruct((R, D), jnp.bfloat16)] * 3,
        compiler_params=pltpu.CompilerParams(
            dimension_semantics=("parallel",),
            vmem_limit_bytes=50 * 1024 * 1024,
        ),
        name="qkv_rmsnorm_rope",
        interpret=interpret,
    )(x2d, wq, wk, wv, g1, cos64, sin64)


def _attn_piece_body(q_ref, k_ref, v_ref, o_ref, *, row0):
    i = pl.program_id(2)
    bq = q_ref.shape[1]
    Sk = k_ref.shape[1]
    q = q_ref[0]
    k = k_ref[0]
    v = v_ref[0]
    rows = row0 + i * bq + jax.lax.broadcasted_iota(jnp.int32, (bq, Sk), 0)
    cols = jax.lax.broadcasted_iota(jnp.int32, (bq, Sk), 1)
    mask = cols <= rows
    halves = []
    for hh in range(2):
        sl = slice(hh * 64, hh * 64 + 64)
        s = jax.lax.dot_general(q[:, sl], k[:, sl], _DN_TB,
                                preferred_element_type=jnp.float32)
        p = jnp.exp2(jnp.where(mask, s, _NEG))
        l = jnp.sum(p, axis=1, keepdims=True)
        oh = jnp.dot(p.astype(jnp.bfloat16), v[:, sl],
                     preferred_element_type=jnp.float32)
        halves.append((oh / l).astype(jnp.bfloat16))
    o_ref[0] = jnp.concatenate(halves, axis=1)


def _attn(q, k, v, *, interpret=False):
    B, S, D = q.shape
    bq = _BQ
    hp = D // 128
    import functools
    pieces = []
    rows_per = S // _NSPLIT
    nq = rows_per // bq
    for p in range(_NSPLIT):
        Sk = (p + 1) * rows_per
        row0 = p * rows_per
        q_spec = pl.BlockSpec((1, bq, 128),
                              lambda b, h, i, _p=p, _nq=nq: (b, _p * _nq + i, h))
        kv_spec = pl.BlockSpec((1, Sk, 128), lambda b, h, i: (b, 0, h))
        o_spec = pl.BlockSpec((1, bq, 128), lambda b, h, i: (b, i, h))
        piece = pl.pallas_call(
            functools.partial(_attn_piece_body, row0=row0),
            grid=(B, hp, nq),
            in_specs=[q_spec, kv_spec, kv_spec],
            out_specs=o_spec,
            out_shape=jax.ShapeDtypeStruct((B, rows_per, D), jnp.bfloat16),
            compiler_params=pltpu.CompilerParams(
                dimension_semantics=("parallel", "arbitrary", "arbitrary"),
                vmem_limit_bytes=40 * 1024 * 1024,
            ),
            name=f"causal_attention_p{p}",
            interpret=interpret,
        )(q, k, v)
        pieces.append(piece)
    return jnp.concatenate(pieces, axis=1)


def _ffn_body(o_ref, x_ref, wo_ref, g2_ref, w1_ref, w3_ref, w2_ref,
              out_ref, h2_ref):
    j = pl.program_id(1)

    @pl.when(j == 0)
    def _():
        x1 = x_ref[...] + jax.lax.dot_general(
            o_ref[...], wo_ref[...], _DN_TB,
            preferred_element_type=jnp.float32)
        out_ref[...] = x1
        ms = jnp.mean(x1 * x1, axis=1, keepdims=True)
        h2_ref[...] = (x1 * jax.lax.rsqrt(ms + _EPS)
                       * g2_ref[...]).astype(jnp.bfloat16)

    h2 = h2_ref[...]
    a = jax.lax.dot_general(h2, w1_ref[...], _DN_TB,
                            preferred_element_type=jnp.float32)
    b = jax.lax.dot_general(h2, w3_ref[...], _DN_TB,
                            preferred_element_type=jnp.float32)
    ff = (a * jax.nn.sigmoid(a) * b).astype(jnp.bfloat16)
    out_ref[...] += jax.lax.dot_general(ff, w2_ref[...], _DN_TB,
                                        preferred_element_type=jnp.float32)


def _ffn(o2d, x2d, wo, g2, w1, w3, w2, *, interpret=False):
    R, D = x2d.shape
    F = w1.shape[0]
    bm = _BM_FFN
    bf = _BF
    nr = R // bm
    nf = F // bf
    row_spec = pl.BlockSpec((bm, D), lambda i, j: (i, 0))
    return pl.pallas_call(
        _ffn_body,
        grid=(nr, nf),
        in_specs=[row_spec, row_spec,
                  pl.BlockSpec((D, D), lambda i, j: (0, 0)),
                  pl.BlockSpec((1, D), lambda i, j: (0, 0)),
                  pl.BlockSpec((bf, D), lambda i, j: (j, 0)),
                  pl.BlockSpec((bf, D), lambda i, j: (j, 0)),
                  pl.BlockSpec((D, bf), lambda i, j: (0, j))],
        out_specs=row_spec,
        out_shape=jax.ShapeDtypeStruct((R, D), jnp.float32),
        scratch_shapes=[pltpu.VMEM((bm, D), jnp.bfloat16)],
        compiler_params=pltpu.CompilerParams(
            dimension_semantics=("parallel", "arbitrary"),
            vmem_limit_bytes=52 * 1024 * 1024,
        ),
        name="outproj_rmsnorm_swiglu",
        interpret=interpret,
    )(o2d, x2d, wo, g2, w1, w3, w2)


def kernel(x, wq, wk, wv, wo, w1, w2, w3, g1, g2, *, interpret=False):
    B, S, D = x.shape
    dh = D // _NUM_HEADS
    x2d = x.reshape(B * S, D)
    wqb = wq.astype(jnp.bfloat16)
    wkb = wk.astype(jnp.bfloat16)
    wvb = wv.astype(jnp.bfloat16)
    wob = wo.astype(jnp.bfloat16)
    w1b = w1.astype(jnp.bfloat16)
    w2b = w2.astype(jnp.bfloat16)
    w3b = w3.astype(jnp.bfloat16)
    inv = _THETA ** (-jnp.arange(0, dh, 2, dtype=jnp.float32) / dh)
    ang = jnp.arange(S, dtype=jnp.float32)[:, None] * inv[None, :]
    cos64 = jnp.repeat(jnp.cos(ang), 2, axis=1)
    sin64 = jnp.repeat(jnp.sin(ang), 2, axis=1)
    q2d, k2d, v2d = _qkv(x2d, wqb, wkb, wvb, g1.reshape(1, D), cos64, sin64,
                         interpret=interpret)
    o = _attn(q2d.reshape(B, S, D), k2d.reshape(B, S, D),
              v2d.reshape(B, S, D), interpret=interpret)
    out2d = _ffn(o.reshape(B * S, D), x2d, wob, g2.reshape(1, D),
                 w1b, w3b, w2b, interpret=interpret)
    return out2d.reshape(B, S, D)


# speedup vs baseline: 1.5221x; 1.0001x over previous
import functools

import jax
import jax.numpy as jnp
from jax.experimental import pallas as pl
from jax.experimental.pallas import tpu as pltpu

_NUM_HEADS = 16
_THETA = 10000.0
_EPS = 1e-5
_NEG = float(jnp.finfo(jnp.float32).min)
_LOG2E = 1.4426950408889634
_DN_TB = (((1,), (1,)), ((), ()))

_BM_QKV = 512
_BQ = 256
_BM_FFN = 512
_BF = 2048
_NSPLIT = 8


def _qkv_body(x_ref, wq_ref, wk_ref, wv_ref, g1_ref, cos_ref, sin_ref,
              q_ref, k_ref, v_ref):
    x = x_ref[...]
    ms = jnp.mean(x * x, axis=1, keepdims=True)
    h = (x * jax.lax.rsqrt(ms + _EPS) * g1_ref[...]).astype(jnp.bfloat16)
    q = jax.lax.dot_general(h, wq_ref[...], _DN_TB,
                            preferred_element_type=jnp.float32)
    k = jax.lax.dot_general(h, wk_ref[...], _DN_TB,
                            preferred_element_type=jnp.float32)
    v = jax.lax.dot_general(h, wv_ref[...], _DN_TB,
                            preferred_element_type=jnp.float32)
    reps = q.shape[1] // cos_ref.shape[1]
    c = cos_ref[...]
    s = sin_ref[...]
    cf = jnp.concatenate([c] * reps, axis=1)
    sf = jnp.concatenate([s] * reps, axis=1)
    lane = jax.lax.broadcasted_iota(jnp.int32, q.shape, 1)
    even = (lane % 2) == 0

    def rope(t):
        fwd = pltpu.roll(t, 1, axis=1)
        bwd = pltpu.roll(t, t.shape[1] - 1, axis=1)
        return t * cf + jnp.where(even, -bwd, fwd) * sf

    q_ref[...] = (rope(q) * (0.125 * _LOG2E)).astype(jnp.bfloat16)
    k_ref[...] = rope(k).astype(jnp.bfloat16)
    v_ref[...] = v.astype(jnp.bfloat16)


def _qkv(x2d, wq, wk, wv, g1, cos64, sin64):
    R, D = x2d.shape
    S = cos64.shape[0]
    bm = _BM_QKV
    nr = R // bm
    npos = S // bm
    w_spec = pl.BlockSpec((D, D), lambda i: (0, 0))
    row_spec = pl.BlockSpec((bm, D), lambda i: (i, 0))
    tab_spec = pl.BlockSpec((bm, cos64.shape[1]), lambda i: (i % npos, 0))
    return pl.pallas_call(
        _qkv_body,
        grid=(nr,),
        in_specs=[row_spec, w_spec, w_spec, w_spec,
                  pl.BlockSpec((1, D), lambda i: (0, 0)),
                  tab_spec, tab_spec],
        out_specs=[row_spec, row_spec, row_spec],
        out_shape=[jax.ShapeDtypeStruct((R, D), jnp.bfloat16)] * 3,
        compiler_params=pltpu.CompilerParams(
            dimension_semantics=("parallel",),
            vmem_limit_bytes=50 * 1024 * 1024,
        ),
        name="qkv_rmsnorm_rope",
    )(x2d, wq, wk, wv, g1, cos64, sin64)


def _attn_piece_body(q_ref, k_ref, v_ref, o_ref, *, row0):
    i = pl.program_id(2)
    bq = q_ref.shape[1]
    Sk = k_ref.shape[1]
    q = q_ref[0]
    k = k_ref[0]
    v = v_ref[0]
    rows = row0 + i * bq + jax.lax.broadcasted_iota(jnp.int32, (bq, Sk), 0)
    cols = jax.lax.broadcasted_iota(jnp.int32, (bq, Sk), 1)
    mask = cols <= rows
    halves = []
    for hh in range(2):
        sl = slice(hh * 64, hh * 64 + 64)
        s = jax.lax.dot_general(q[:, sl], k[:, sl], _DN_TB,
                                preferred_element_type=jnp.float32)
        p = jnp.exp2(jnp.where(mask, s, _NEG))
        l = jnp.sum(p, axis=1, keepdims=True)
        oh = jnp.dot(p.astype(jnp.bfloat16), v[:, sl],
                     preferred_element_type=jnp.float32)
        halves.append((oh / l).astype(jnp.bfloat16))
    o_ref[0] = jnp.concatenate(halves, axis=1)


def _attn(q, k, v):
    B, S, D = q.shape
    bq = _BQ
    hp = D // 128
    pieces = []
    rows_per = S // _NSPLIT
    nq = rows_per // bq
    for p in range(_NSPLIT):
        Sk = (p + 1) * rows_per
        row0 = p * rows_per
        q_spec = pl.BlockSpec((1, bq, 128),
                              lambda b, h, i, _p=p, _nq=nq: (b, _p * _nq + i, h))
        kv_spec = pl.BlockSpec((1, Sk, 128), lambda b, h, i: (b, 0, h))
        o_spec = pl.BlockSpec((1, bq, 128), lambda b, h, i: (b, i, h))
        piece = pl.pallas_call(
            functools.partial(_attn_piece_body, row0=row0),
            grid=(B, hp, nq),
            in_specs=[q_spec, kv_spec, kv_spec],
            out_specs=o_spec,
            out_shape=jax.ShapeDtypeStruct((B, rows_per, D), jnp.bfloat16),
            compiler_params=pltpu.CompilerParams(
                dimension_semantics=("parallel", "arbitrary", "arbitrary"),
                vmem_limit_bytes=40 * 1024 * 1024,
            ),
            name=f"causal_attention_p{p}",
        )(q, k, v)
        pieces.append(piece)
    return jnp.concatenate(pieces, axis=1)


def _ffn_body(o_ref, x_ref, wo_ref, g2_ref, w1_ref, w3_ref, w2_ref,
              out_ref, h2_ref):
    j = pl.program_id(1)

    @pl.when(j == 0)
    def _():
        x1 = x_ref[...] + jax.lax.dot_general(
            o_ref[...], wo_ref[...], _DN_TB,
            preferred_element_type=jnp.float32)
        out_ref[...] = x1
        ms = jnp.mean(x1 * x1, axis=1, keepdims=True)
        h2_ref[...] = (x1 * jax.lax.rsqrt(ms + _EPS)
                       * g2_ref[...]).astype(jnp.bfloat16)

    h2 = h2_ref[...]
    a = jax.lax.dot_general(h2, w1_ref[...], _DN_TB,
                            preferred_element_type=jnp.float32)
    b = jax.lax.dot_general(h2, w3_ref[...], _DN_TB,
                            preferred_element_type=jnp.float32)
    ff = (a * jax.nn.sigmoid(a) * b).astype(jnp.bfloat16)
    out_ref[...] += jax.lax.dot_general(ff, w2_ref[...], _DN_TB,
                                        preferred_element_type=jnp.float32)


def _ffn(o2d, x2d, wo, g2, w1, w3, w2):
    R, D = x2d.shape
    F = w1.shape[0]
    bm = _BM_FFN
    bf = _BF
    nr = R // bm
    nf = F // bf
    row_spec = pl.BlockSpec((bm, D), lambda i, j: (i, 0))
    return pl.pallas_call(
        _ffn_body,
        grid=(nr, nf),
        in_specs=[row_spec, row_spec,
                  pl.BlockSpec((D, D), lambda i, j: (0, 0)),
                  pl.BlockSpec((1, D), lambda i, j: (0, 0)),
                  pl.BlockSpec((bf, D), lambda i, j: (j, 0)),
                  pl.BlockSpec((bf, D), lambda i, j: (j, 0)),
                  pl.BlockSpec((D, bf), lambda i, j: (0, j))],
        out_specs=row_spec,
        out_shape=jax.ShapeDtypeStruct((R, D), jnp.float32),
        scratch_shapes=[pltpu.VMEM((bm, D), jnp.bfloat16)],
        compiler_params=pltpu.CompilerParams(
            dimension_semantics=("parallel", "arbitrary"),
            vmem_limit_bytes=52 * 1024 * 1024,
        ),
        name="outproj_rmsnorm_swiglu",
    )(o2d, x2d, wo, g2, w1, w3, w2)


def kernel(x, wq, wk, wv, wo, w1, w2, w3, g1, g2):
    B, S, D = x.shape
    dh = D // _NUM_HEADS
    x2d = x.reshape(B * S, D)
    wqb = wq.astype(jnp.bfloat16)
    wkb = wk.astype(jnp.bfloat16)
    wvb = wv.astype(jnp.bfloat16)
    wob = wo.astype(jnp.bfloat16)
    w1b = w1.astype(jnp.bfloat16)
    w2b = w2.astype(jnp.bfloat16)
    w3b = w3.astype(jnp.bfloat16)
    inv = _THETA ** (-jnp.arange(0, dh, 2, dtype=jnp.float32) / dh)
    ang = jnp.arange(S, dtype=jnp.float32)[:, None] * inv[None, :]
    cos64 = jnp.repeat(jnp.cos(ang), 2, axis=1)
    sin64 = jnp.repeat(jnp.sin(ang), 2, axis=1)
    q2d, k2d, v2d = _qkv(x2d, wqb, wkb, wvb, g1.reshape(1, D), cos64, sin64)
    o = _attn(q2d.reshape(B, S, D), k2d.reshape(B, S, D),
              v2d.reshape(B, S, D))
    out2d = _ffn(o.reshape(B * S, D), x2d, wob, g2.reshape(1, D),
                 w1b, w3b, w2b)
    return out2d.reshape(B, S, D)
